# baseline (speedup 1.0000x reference)
import functools

import jax
import jax.numpy as jnp
from jax import lax
from jax.experimental import pallas as pl
from jax.experimental.pallas import tpu as pltpu
from jax.experimental.pallas import tpu_sc as plsc

_NC = 2
_NS = 16
_NW = _NC * _NS


@functools.lru_cache(maxsize=None)
def _build_gather(B: int, V: int, D: int):
    assert B % _NW == 0
    b_per_w = B // _NW
    n_chunks = 16
    assert b_per_w % n_chunks == 0
    C = b_per_w // n_chunks
    assert C % 8 == 0

    mesh = plsc.VectorSubcoreMesh(core_axis_name="c", subcore_axis_name="s")

    @functools.partial(
        pl.kernel,
        mesh=mesh,
        out_type=jax.ShapeDtypeStruct((B, D), jnp.float32),
        scratch_types=[
            pltpu.VMEM((b_per_w,), jnp.int32),
            pltpu.VMEM((C, D), jnp.float32),
            pltpu.VMEM((C, D), jnp.float32),
            pltpu.SemaphoreType.DMA,
            pltpu.SemaphoreType.DMA,
            pltpu.SemaphoreType.DMA,
            pltpu.SemaphoreType.DMA,
        ],
        compiler_params=pltpu.CompilerParams(use_tc_tiling_on_sc=False),
    )
    def gather_kernel(idx_hbm, table_hbm, out_hbm, idx_v, rows0, rows1,
                      g0, g1, w0, w1):
        wid = lax.axis_index("s") * _NC + lax.axis_index("c")
        base = wid * b_per_w
        rows = (rows0, rows1)
        gsem = (g0, g1)
        wsem = (w0, w1)
        pltpu.sync_copy(idx_hbm.at[pl.ds(base, b_per_w)], idx_v)

        def gather(c):
            return pltpu.async_copy(
                table_hbm.at[idx_v.at[pl.ds(c * C, C)]], rows[c % 2],
                gsem[c % 2])

        def write(c):
            return pltpu.async_copy(
                rows[c % 2], out_hbm.at[pl.ds(base + c * C, C)], wsem[c % 2])

        gathers = [gather(0)]
        writes = [None, None]
        for c in range(n_chunks):
            gathers[c].wait()
            if c + 1 < n_chunks:
                if writes[(c + 1) % 2] is not None:
                    writes[(c + 1) % 2].wait()
                gathers.append(gather(c + 1))
            writes[c % 2] = write(c)
        writes[(n_chunks - 1) % 2].wait()
        if n_chunks > 1:
            writes[(n_chunks - 2) % 2].wait()

    return gather_kernel


def kernel(x, weight):
    Bm, F = x.shape
    V, D = weight.shape
    B = Bm * F
    xf = x.reshape(B).astype(jnp.int32)
    out = _build_gather(B, V, D)(xf, weight)
    return out.reshape(Bm, F, D)
